# Initial kernel scaffold; baseline (speedup 1.0000x reference)
import jax
import jax.numpy as jnp
from jax.experimental import pallas as pl


def kernel(x):
    raise NotImplementedError("write your pallas kernel here")


# speedup vs baseline: 2.3491x; 2.3491x over previous
import functools

import jax
import jax.numpy as jnp
from jax import lax
from jax.experimental import pallas as pl
from jax.experimental.pallas import tpu as pltpu
from jax.experimental.pallas import tpu_sc as plsc

NC = 2
NS = 16
NW = NC * NS
L = 16

H = 224
W = 224
OUT = H // 2
BC = 4 * 96
ROWS = BC * OUT
RPW = ROWS // NW
RB = 32
NBLK = RPW // RB
GROUPS = OUT // L


def _fuzzy_body(x_hbm, o_hbm, in_v, out_v):
    wid = lax.axis_index("s") * NC + lax.axis_index("c")
    iota = lax.iota(jnp.int32, L)
    col0 = iota * 2

    def blk_body(b, carry):
        base = wid * RPW + b * RB
        pltpu.sync_copy(x_hbm.at[pl.ds(base * (2 * W), RB * 2 * W)], in_v)

        def row_body(r, c2):
            b0 = r * (2 * W)
            for g in range(GROUPS):
                i0 = b0 + g * (2 * L) + col0
                x0 = plsc.load_gather(in_v, [i0])
                x1 = plsc.load_gather(in_v, [i0 + 1])
                x2 = plsc.load_gather(in_v, [i0 + W])
                x3 = plsc.load_gather(in_v, [i0 + (W + 1)])

                m_inner = (x1 + x2) * 0.5
                m_all = ((x0 + x1) + (x2 + x3)) * 0.25
                v_avg = (m_inner + m_all) * 0.5
                w0 = jnp.abs(x0 - v_avg)
                w1 = jnp.abs(x1 - v_avg)
                w2 = jnp.abs(x2 - v_avg)
                w3 = jnp.abs(x3 - v_avg)
                s0 = (w1 + w2) * 0.5 + 1e-4
                s1 = ((w0 + w1) + (w2 + w3)) * 0.25 + 1e-4

                def gauss(xm, mu, s):
                    z = (xm - mu) / s
                    return jnp.exp(z * z * -0.5)

                p00 = gauss(x0, m_inner, s0)
                p01 = gauss(x1, m_inner, s0)
                p02 = gauss(x2, m_inner, s0)
                p03 = gauss(x3, m_inner, s0)
                p10 = gauss(x0, m_all, s1)
                p11 = gauss(x1, m_all, s1)
                p12 = gauss(x2, m_all, s1)
                p13 = gauss(x3, m_all, s1)

                thresh = jnp.minimum(
                    jnp.minimum(jnp.maximum(p00, p10), jnp.maximum(p01, p11)),
                    jnp.minimum(jnp.maximum(p02, p12), jnp.maximum(p03, p13)))
                a0 = (p00 + p10) * 0.5
                a1 = (p01 + p11) * 0.5
                a2 = (p02 + p12) * 0.5
                a3 = (p03 + p13) * 0.5

                primary = a1 >= thresh
                secondary = jnp.logical_and(jnp.logical_not(primary),
                                            s1 < 0.001)
                num = (a0 * x0 + a1 * x1) + (a2 * x2 + a3 * x3)
                den = (a0 + a1) + (a2 + a3)
                denoised = num / den
                res = jnp.where(primary, m_all,
                                jnp.where(secondary, v_avg, denoised))
                out_v[pl.ds(r * OUT + g * L, L)] = res
            return c2

        lax.fori_loop(0, RB, row_body, 0)
        pltpu.sync_copy(out_v, o_hbm.at[pl.ds(base * OUT, RB * OUT)])
        return carry

    lax.fori_loop(0, NBLK, blk_body, 0)


_mesh = plsc.VectorSubcoreMesh(core_axis_name="c", subcore_axis_name="s",
                               num_cores=NC, num_subcores=NS)

_fuzzy_call = pl.kernel(
    _fuzzy_body,
    out_type=jax.ShapeDtypeStruct((ROWS * OUT,), jnp.float32),
    mesh=_mesh,
    scratch_types=[
        pltpu.VMEM((RB * 2 * W,), jnp.float32),
        pltpu.VMEM((RB * OUT,), jnp.float32),
    ],
    compiler_params=pltpu.CompilerParams(needs_layout_passes=False),
)


@jax.jit
def kernel(x):
    B, C, _, _ = x.shape
    out_flat = _fuzzy_call(x.reshape(-1))
    return out_flat.reshape(B, C, OUT, OUT)


# speedup vs baseline: 2.7157x; 1.1561x over previous
import functools

import jax
import jax.numpy as jnp
from jax import lax
from jax.experimental import pallas as pl
from jax.experimental.pallas import tpu as pltpu
from jax.experimental.pallas import tpu_sc as plsc

NC = 2
NS = 16
NW = NC * NS
L = 16

H = 224
W = 224
OUT = H // 2
BC = 4 * 96
ROWS = BC * OUT
RPW = ROWS // NW
RB = 32
NBLK = RPW // RB
GROUPS = OUT // L


def _fuzzy_body(x_hbm, o_hbm, in_v, out_v):
    wid = lax.axis_index("s") * NC + lax.axis_index("c")
    iota = lax.iota(jnp.int32, L)
    col0 = iota * 2

    def blk_body(b, carry):
        base = wid * RPW + b * RB
        pltpu.sync_copy(x_hbm.at[pl.ds(base * (2 * W), RB * 2 * W)], in_v)

        @plsc.parallel_loop(0, RB, unroll=2)
        def row_body(r):
            b0 = r * (2 * W)
            for g in range(GROUPS):
                i0 = b0 + g * (2 * L) + col0
                x0 = plsc.load_gather(in_v, [i0])
                x1 = plsc.load_gather(in_v, [i0 + 1])
                x2 = plsc.load_gather(in_v, [i0 + W])
                x3 = plsc.load_gather(in_v, [i0 + (W + 1)])

                m_inner = (x1 + x2) * 0.5
                m_all = ((x0 + x1) + (x2 + x3)) * 0.25
                v_avg = (m_inner + m_all) * 0.5
                w0 = jnp.abs(x0 - v_avg)
                w1 = jnp.abs(x1 - v_avg)
                w2 = jnp.abs(x2 - v_avg)
                w3 = jnp.abs(x3 - v_avg)
                s0 = (w1 + w2) * 0.5 + 1e-4
                s1 = ((w0 + w1) + (w2 + w3)) * 0.25 + 1e-4

                r0 = 1.0 / s0
                r1 = 1.0 / s1
                c0 = (r0 * r0) * -0.5
                c1 = (r1 * r1) * -0.5

                def gauss(xm, mu, c):
                    d = xm - mu
                    return jnp.exp((d * d) * c)

                p00 = gauss(x0, m_inner, c0)
                p01 = gauss(x1, m_inner, c0)
                p02 = gauss(x2, m_inner, c0)
                p03 = gauss(x3, m_inner, c0)
                p10 = gauss(x0, m_all, c1)
                p11 = gauss(x1, m_all, c1)
                p12 = gauss(x2, m_all, c1)
                p13 = gauss(x3, m_all, c1)

                thresh = jnp.minimum(
                    jnp.minimum(jnp.maximum(p00, p10), jnp.maximum(p01, p11)),
                    jnp.minimum(jnp.maximum(p02, p12), jnp.maximum(p03, p13)))
                q0 = p00 + p10
                q1 = p01 + p11
                q2 = p02 + p12
                q3 = p03 + p13

                primary = q1 >= (thresh + thresh)
                secondary = jnp.logical_and(jnp.logical_not(primary),
                                            s1 < 0.001)
                num = (q0 * x0 + q1 * x1) + (q2 * x2 + q3 * x3)
                den = (q0 + q1) + (q2 + q3)
                denoised = num / den
                res = jnp.where(primary, m_all,
                                jnp.where(secondary, v_avg, denoised))
                out_v[pl.ds(r * OUT + g * L, L)] = res
        pltpu.sync_copy(out_v, o_hbm.at[pl.ds(base * OUT, RB * OUT)])
        return carry

    lax.fori_loop(0, NBLK, blk_body, 0)


_mesh = plsc.VectorSubcoreMesh(core_axis_name="c", subcore_axis_name="s",
                               num_cores=NC, num_subcores=NS)

_fuzzy_call = pl.kernel(
    _fuzzy_body,
    out_type=jax.ShapeDtypeStruct((ROWS * OUT,), jnp.float32),
    mesh=_mesh,
    scratch_types=[
        pltpu.VMEM((RB * 2 * W,), jnp.float32),
        pltpu.VMEM((RB * OUT,), jnp.float32),
    ],
    compiler_params=pltpu.CompilerParams(needs_layout_passes=False),
)


@jax.jit
def kernel(x):
    B, C, _, _ = x.shape
    out_flat = _fuzzy_call(x.reshape(-1))
    return out_flat.reshape(B, C, OUT, OUT)
